# Initial kernel scaffold; baseline (speedup 1.0000x reference)
import jax
import jax.numpy as jnp
from jax.experimental import pallas as pl


def kernel(z, embedding_weight):
    raise NotImplementedError("write your pallas kernel here")


# speedup vs baseline: 1.1574x; 1.1574x over previous
import functools

import jax
import jax.numpy as jnp
from jax.experimental import pallas as pl
from jax.experimental.pallas import tpu as pltpu

B = 16
D = 64
T = 1024
K = 1024
BETA = 0.25
EPS = 1e-10


def _vq_step(z_ref, e_ref, esq_ref, zq_ref, loss_ref, perp_ref,
             onehot_ref, idx_ref, idx_acc, loss_acc):
    b = pl.program_id(0)
    zb = z_ref[0]
    emb = e_ref[...]
    esq = esq_ref[...]

    zf = zb.T
    mm = jax.lax.dot_general(
        zf, emb, (((1,), (1,)), ((), ())),
        preferred_element_type=jnp.float32)
    zsq = jnp.sum(zf * zf, axis=1, keepdims=True)
    dist = (zsq + esq) - 2.0 * mm

    dmin = jnp.min(dist, axis=1, keepdims=True)
    iota_k = jax.lax.broadcasted_iota(jnp.int32, (T, K), 1)
    idx = jnp.min(jnp.where(dist == dmin, iota_k, K),
                  axis=1, keepdims=True)

    onehot = (iota_k == idx).astype(jnp.float32)
    onehot_ref[0] = onehot
    idx_ref[0] = idx.T
    idx_acc[pl.ds(b, 1), :] = idx.T

    zq = jax.lax.dot_general(
        onehot, emb, (((1,), (0,)), ((), ())),
        preferred_element_type=jnp.float32,
        precision=jax.lax.Precision.HIGHEST)
    zq_t = zq.T
    zq_ref[0] = zq_t

    diff = zb - zq_t
    part = jnp.sum(diff * diff).reshape(1, 1)

    @pl.when(b == 0)
    def _init():
        loss_acc[...] = part

    @pl.when(b > 0)
    def _accum():
        loss_acc[...] = loss_acc[...] + part

    @pl.when(b == B - 1)
    def _finalize():
        mse = loss_acc[...] * (1.0 / (B * D * T))
        loss_ref[...] = mse + BETA * mse

        all_idx = idx_acc[...]
        m = jnp.zeros((B, T), jnp.float32)
        for bb in range(B):
            m = m + (all_idx == all_idx[bb:bb + 1, :]).astype(jnp.float32)
        ent_sum = jnp.sum(jnp.log(m * (1.0 / B) + EPS))
        perp_ref[...] = jnp.exp(-(1.0 / B) * ent_sum).reshape(1, 1)


@jax.jit
def kernel(z, embedding_weight):
    esq = jnp.sum(embedding_weight ** 2, axis=1).reshape(1, K)
    zq, loss, perp, onehot, idx = pl.pallas_call(
        _vq_step,
        grid=(B,),
        in_specs=[
            pl.BlockSpec((1, D, T), lambda b: (b, 0, 0)),
            pl.BlockSpec((K, D), lambda b: (0, 0)),
            pl.BlockSpec((1, K), lambda b: (0, 0)),
        ],
        out_specs=[
            pl.BlockSpec((1, D, T), lambda b: (b, 0, 0)),
            pl.BlockSpec((1, 1), lambda b: (0, 0)),
            pl.BlockSpec((1, 1), lambda b: (0, 0)),
            pl.BlockSpec((1, T, K), lambda b: (b, 0, 0)),
            pl.BlockSpec((1, 1, T), lambda b: (b, 0, 0)),
        ],
        out_shape=[
            jax.ShapeDtypeStruct((B, D, T), jnp.float32),
            jax.ShapeDtypeStruct((1, 1), jnp.float32),
            jax.ShapeDtypeStruct((1, 1), jnp.float32),
            jax.ShapeDtypeStruct((B, T, K), jnp.float32),
            jax.ShapeDtypeStruct((B, 1, T), jnp.int32),
        ],
        scratch_shapes=[
            pltpu.VMEM((B, T), jnp.int32),
            pltpu.VMEM((1, 1), jnp.float32),
        ],
    )(z, embedding_weight, esq)
    return (zq, loss.reshape(()), perp.reshape(()), onehot,
            idx.reshape(B, T))


# speedup vs baseline: 1.9483x; 1.6833x over previous
import functools

import jax
import jax.numpy as jnp
from jax.experimental import pallas as pl
from jax.experimental.pallas import tpu as pltpu

B = 16
D = 64
T = 1024
K = 1024
BETA = 0.25
EPS = 1e-10


def _vq_step(z_ref, e_ref, esq_ref, zq_ref, loss_ref, perp_ref,
             onehot_ref, idx_ref, idx_acc, loss_acc):
    b = pl.program_id(0)
    zb = z_ref[0]
    emb = e_ref[...]
    esq = esq_ref[...]

    zf = zb.T
    mm = jax.lax.dot_general(
        zf, emb, (((1,), (1,)), ((), ())),
        preferred_element_type=jnp.float32)
    zsq = jnp.sum(zf * zf, axis=1, keepdims=True)
    dist = (zsq + esq) - 2.0 * mm

    dmin = jnp.min(dist, axis=1, keepdims=True)
    iota_k = jax.lax.broadcasted_iota(jnp.int32, (T, K), 1)
    idx = jnp.min(jnp.where(dist == dmin, iota_k, K),
                  axis=1, keepdims=True)

    onehot = (iota_k == idx).astype(jnp.float32)
    onehot_ref[0] = onehot
    idx_ref[0] = idx.T
    idx_acc[pl.ds(b, 1), :] = idx.T

    zq = jax.lax.dot_general(
        onehot, emb, (((1,), (0,)), ((), ())),
        preferred_element_type=jnp.float32)
    zq_t = zq.T
    zq_ref[0] = zq_t

    diff = zb - zq_t
    part = jnp.sum(diff * diff).reshape(1, 1)

    @pl.when(b == 0)
    def _init():
        loss_acc[...] = part

    @pl.when(b > 0)
    def _accum():
        loss_acc[...] = loss_acc[...] + part

    @pl.when(b == B - 1)
    def _finalize():
        mse = loss_acc[...] * (1.0 / (B * D * T))
        loss_ref[...] = mse + BETA * mse

        all_idx = idx_acc[...]
        m = jnp.zeros((B, T), jnp.float32)
        for bb in range(B):
            m = m + (all_idx == all_idx[bb:bb + 1, :]).astype(jnp.float32)
        ent_sum = jnp.sum(jnp.log(m * (1.0 / B) + EPS))
        perp_ref[...] = jnp.exp(-(1.0 / B) * ent_sum).reshape(1, 1)


@jax.jit
def kernel(z, embedding_weight):
    esq = jnp.sum(embedding_weight ** 2, axis=1).reshape(1, K)
    zq, loss, perp, onehot, idx = pl.pallas_call(
        _vq_step,
        grid=(B,),
        in_specs=[
            pl.BlockSpec((1, D, T), lambda b: (b, 0, 0)),
            pl.BlockSpec((K, D), lambda b: (0, 0)),
            pl.BlockSpec((1, K), lambda b: (0, 0)),
        ],
        out_specs=[
            pl.BlockSpec((1, D, T), lambda b: (b, 0, 0)),
            pl.BlockSpec((1, 1), lambda b: (0, 0)),
            pl.BlockSpec((1, 1), lambda b: (0, 0)),
            pl.BlockSpec((1, T, K), lambda b: (b, 0, 0)),
            pl.BlockSpec((1, 1, T), lambda b: (b, 0, 0)),
        ],
        out_shape=[
            jax.ShapeDtypeStruct((B, D, T), jnp.float32),
            jax.ShapeDtypeStruct((1, 1), jnp.float32),
            jax.ShapeDtypeStruct((1, 1), jnp.float32),
            jax.ShapeDtypeStruct((B, T, K), jnp.float32),
            jax.ShapeDtypeStruct((B, 1, T), jnp.int32),
        ],
        scratch_shapes=[
            pltpu.VMEM((B, T), jnp.int32),
            pltpu.VMEM((1, 1), jnp.float32),
        ],
    )(z, embedding_weight, esq)
    return (zq, loss.reshape(()), perp.reshape(()), onehot,
            idx.reshape(B, T))
